# baseline (speedup 1.0000x reference)
import functools

import jax
import jax.numpy as jnp
from jax import lax
from jax.experimental import pallas as pl
from jax.experimental.pallas import tpu as pltpu
from jax.experimental.pallas import tpu_sc as plsc

_NC = 2

_CHUNK = 16
_NBUF = 24
_DEPTH = 12


def _sc_body(rows_per_w, emb_hbm, out_hbm, sbuf, in_sems, out_sems):
    base = lax.axis_index("c") * rows_per_w
    nchunks = rows_per_w // _CHUNK

    def in_copy(c, b):
        return pltpu.make_async_copy(
            emb_hbm.at[pl.ds(base + c * _CHUNK, _CHUNK)], sbuf.at[b],
            in_sems.at[b])

    def out_copy(c, b):
        return pltpu.make_async_copy(
            sbuf.at[b], out_hbm.at[pl.ds(base + c * _CHUNK, _CHUNK)],
            out_sems.at[b])

    for c in range(min(_DEPTH, nchunks)):
        in_copy(c, c % _NBUF).start()
    for c in range(nchunks):
        b = c % _NBUF
        p = c + _DEPTH
        if p < nchunks:
            bp = p % _NBUF
            if p - _NBUF >= 0:
                out_copy(p - _NBUF, bp).wait()
            in_copy(p, bp).start()
        in_copy(c, b).wait()
        out_copy(c, b).start()
    for c in range(max(0, nchunks - _NBUF), nchunks):
        out_copy(c, c % _NBUF).wait()


def kernel(x, embeddings):
    seq_len = x.shape[1]
    hidden = embeddings.shape[1]
    rows_per_w = seq_len // _NC
    mesh = plsc.ScalarSubcoreMesh(axis_name="c", num_cores=_NC)
    sc_copy = functools.partial(
        pl.kernel,
        mesh=mesh,
        out_type=jax.ShapeDtypeStruct((seq_len, hidden), embeddings.dtype),
        scratch_types=[
            pltpu.VMEM_SHARED((_NBUF, _CHUNK, hidden), embeddings.dtype),
            pltpu.SemaphoreType.DMA((_NBUF,)),
            pltpu.SemaphoreType.DMA((_NBUF,)),
        ],
    )(functools.partial(_sc_body, rows_per_w))
    return sc_copy(embeddings[:seq_len])


# speedup vs baseline: 1.0011x; 1.0011x over previous
import functools

import jax
from jax import lax
from jax.experimental import pallas as pl
from jax.experimental.pallas import tpu as pltpu
from jax.experimental.pallas import tpu_sc as plsc

_NC = 2

_CHUNK = 32
_NBUF = 12
_DEPTH = 6


def _sc_body(rows_per_w, emb_hbm, out_hbm, sbuf, in_sems, out_sems):
    base = lax.axis_index("c") * rows_per_w
    nchunks = rows_per_w // _CHUNK

    def in_copy(c, b):
        return pltpu.make_async_copy(
            emb_hbm.at[pl.ds(base + c * _CHUNK, _CHUNK)], sbuf.at[b],
            in_sems.at[b])

    def out_copy(c, b):
        return pltpu.make_async_copy(
            sbuf.at[b], out_hbm.at[pl.ds(base + c * _CHUNK, _CHUNK)],
            out_sems.at[b])

    for c in range(min(_DEPTH, nchunks)):
        in_copy(c, c % _NBUF).start()
    for c in range(nchunks):
        b = c % _NBUF
        p = c + _DEPTH
        if p < nchunks:
            bp = p % _NBUF
            if p - _NBUF >= 0:
                out_copy(p - _NBUF, bp).wait()
            in_copy(p, bp).start()
        in_copy(c, b).wait()
        out_copy(c, b).start()
    for c in range(max(0, nchunks - _NBUF), nchunks):
        out_copy(c, c % _NBUF).wait()


def kernel(x, embeddings):
    seq_len = x.shape[1]
    hidden = embeddings.shape[1]
    rows_per_w = seq_len // _NC
    mesh = plsc.ScalarSubcoreMesh(axis_name="c", num_cores=_NC)
    sc_copy = functools.partial(
        pl.kernel,
        mesh=mesh,
        out_type=jax.ShapeDtypeStruct((seq_len, hidden), embeddings.dtype),
        scratch_types=[
            pltpu.VMEM_SHARED((_NBUF, _CHUNK, hidden), embeddings.dtype),
            pltpu.SemaphoreType.DMA((_NBUF,)),
            pltpu.SemaphoreType.DMA((_NBUF,)),
        ],
    )(functools.partial(_sc_body, rows_per_w))
    return sc_copy(embeddings[:seq_len])


# speedup vs baseline: 1.0022x; 1.0010x over previous
import functools

import jax
from jax import lax
from jax.experimental import pallas as pl
from jax.experimental.pallas import tpu as pltpu
from jax.experimental.pallas import tpu_sc as plsc

_NC = 2

_CHUNK = 32
_NBUF = 12
_DEPTH = 6


def _sc_body(rows_per_w, emb_hbm, out_hbm, sbuf, in_sems, out_sems):
    cid = lax.axis_index("c")
    nchunks = rows_per_w // _CHUNK

    def row0(c):
        return (2 * c + cid) * _CHUNK

    def in_copy(c, b):
        return pltpu.make_async_copy(
            emb_hbm.at[pl.ds(row0(c), _CHUNK)], sbuf.at[b], in_sems.at[b])

    def out_copy(c, b):
        return pltpu.make_async_copy(
            sbuf.at[b], out_hbm.at[pl.ds(row0(c), _CHUNK)], out_sems.at[b])

    for c in range(min(_DEPTH, nchunks)):
        in_copy(c, c % _NBUF).start()
    for c in range(nchunks):
        b = c % _NBUF
        p = c + _DEPTH
        if p < nchunks:
            bp = p % _NBUF
            if p - _NBUF >= 0:
                out_copy(p - _NBUF, bp).wait()
            in_copy(p, bp).start()
        in_copy(c, b).wait()
        out_copy(c, b).start()
    for c in range(max(0, nchunks - _NBUF), nchunks):
        out_copy(c, c % _NBUF).wait()


def kernel(x, embeddings):
    seq_len = x.shape[1]
    hidden = embeddings.shape[1]
    rows_per_w = seq_len // _NC
    mesh = plsc.ScalarSubcoreMesh(axis_name="c", num_cores=_NC)
    sc_copy = functools.partial(
        pl.kernel,
        mesh=mesh,
        out_type=jax.ShapeDtypeStruct((seq_len, hidden), embeddings.dtype),
        scratch_types=[
            pltpu.VMEM_SHARED((_NBUF, _CHUNK, hidden), embeddings.dtype),
            pltpu.SemaphoreType.DMA((_NBUF,)),
            pltpu.SemaphoreType.DMA((_NBUF,)),
        ],
    )(functools.partial(_sc_body, rows_per_w))
    return sc_copy(embeddings[:seq_len])
